# baseline (speedup 1.0000x reference)
import jax
import jax.numpy as jnp
from jax import lax
from jax.experimental import pallas as pl
from jax.experimental.pallas import tpu as pltpu
from jax.experimental.pallas import tpu_sc as plsc

B = 16384
F = 26
FIELD = 100000
TOTAL = F * FIELD
K = 16

NC = 2
NS = 16
NW = NC * NS
ROWS_PER_W = B // NW
CHUNK = 128
NCHUNK = ROWS_PER_W // CHUNK
NBUF = 2


SB = 1024
NSB = TOTAL // SB
NTAIL = TOTAL - NSB * SB


def _tr_body(wt_hbm, wtail_hbm, out_hbm, xb0, xb1, yb0, yb1,
             si0, si1, so0, so1):
    wid = lax.axis_index("s") * NC + lax.axis_index("c")
    lanes = lax.iota(jnp.int32, 16)
    nfull = NSB // NW + jnp.where(wid < NSB % NW, 1, 0)

    def start_in(b, xb, si):
        pltpu.async_copy(wt_hbm.at[pl.ds(0, 8), pl.ds(b * SB, SB)],
                         xb.at[pl.ds(0, 8), :], si)
        pltpu.async_copy(wt_hbm.at[pl.ds(8, 8), pl.ds(b * SB, SB)],
                         xb.at[pl.ds(8, 8), :], si)

    def wait_in(xb, si):
        pltpu.make_async_copy(wt_hbm.at[:, pl.ds(0, SB)], xb, si).wait()

    def wait_out(yb, so):
        pltpu.make_async_copy(out_hbm.at[pl.ds(0, SB // 8), :], yb, so).wait()

    def permute(xb, yb, j0, nj8):
        def col_grp(j8, jv):
            for jj in range(8):
                col = plsc.load_gather(xb, [lanes, jv])
                yb[j8, pl.ds(jj * 16, 16)] = col
                jv = jv + 1
            return jv

        lax.fori_loop(0, nj8, col_grp, jnp.broadcast_to(j0, (16,)))

    npair = nfull // 2
    odd = nfull - npair * 2

    start_in(wid, xb0, si0)

    def pair_body(i2, carry):
        b0 = wid + (2 * i2) * NW
        start_in(b0 + NW, xb1, si1)

        @pl.when(i2 >= 1)
        def _():
            wait_out(yb0, so0)

        wait_in(xb0, si0)
        permute(xb0, yb0, 0, SB // 8)
        pltpu.async_copy(yb0, out_hbm.at[pl.ds(b0 * (SB // 8), SB // 8), :],
                         so0)

        @pl.when(2 * i2 + 2 < nfull)
        def _():
            start_in(b0 + 2 * NW, xb0, si0)

        @pl.when(i2 >= 1)
        def _():
            wait_out(yb1, so1)

        wait_in(xb1, si1)
        permute(xb1, yb1, 0, SB // 8)
        pltpu.async_copy(yb1,
                         out_hbm.at[pl.ds((b0 + NW) * (SB // 8), SB // 8), :],
                         so1)
        return carry

    @pl.when(npair >= 1)
    def _():
        lax.fori_loop(0, npair, pair_body, 0)
        wait_out(yb0, so0)
        wait_out(yb1, so1)

    @pl.when(odd == 1)
    def _():
        b = wid + (nfull - 1) * NW
        wait_in(xb0, si0)
        permute(xb0, yb0, 0, SB // 8)
        pltpu.async_copy(yb0, out_hbm.at[pl.ds(b * (SB // 8), SB // 8), :],
                         so0).wait()

    @pl.when(wid == NW - 1)
    def _tail():
        pltpu.async_copy(wtail_hbm, xb0.at[:, pl.ds(0, 128)], si0).wait()
        permute(xb0, yb0, 64, NTAIL // 8)
        pltpu.async_copy(yb0.at[pl.ds(0, NTAIL // 8), :],
                         out_hbm.at[pl.ds(NSB * (SB // 8), NTAIL // 8), :],
                         so0).wait()


@jax.jit
def _tr_call(wt, wtail):
    mesh = plsc.VectorSubcoreMesh(core_axis_name="c", subcore_axis_name="s")
    return pl.kernel(
        _tr_body,
        mesh=mesh,
        compiler_params=pltpu.CompilerParams(
            needs_layout_passes=False, use_tc_tiling_on_sc=True),
        out_type=jax.ShapeDtypeStruct((TOTAL * K // 128, 128), jnp.float32),
        scratch_types=[
            pltpu.VMEM((16, SB), jnp.float32),
            pltpu.VMEM((16, SB), jnp.float32),
            pltpu.VMEM((SB // 8, 128), jnp.float32),
            pltpu.VMEM((SB // 8, 128), jnp.float32),
            pltpu.SemaphoreType.DMA,
            pltpu.SemaphoreType.DMA,
            pltpu.SemaphoreType.DMA,
            pltpu.SemaphoreType.DMA,
        ],
    )(wt, wtail)


def _fm_body(idx_hbm, wso_hbm, wlin_hbm, bias_hbm, out_hbm,
             xTc0, xTc1, rows, lin, tb, outb, biasv, sem_idx, sem_rows, sem_lin):
    xTc = (xTc0, xTc1)
    wid = lax.axis_index("s") * NC + lax.axis_index("c")
    base = wid * ROWS_PER_W

    pltpu.sync_copy(bias_hbm, biasv)
    bv = biasv[...]

    def fetch(c):
        buf = c % NBUF
        pltpu.sync_copy(idx_hbm.at[wid, c], xTc[buf])
        h_rows = pltpu.async_copy(wso_hbm.at[xTc[buf]], rows.at[buf], sem_rows)
        h_lin = pltpu.async_copy(wlin_hbm.at[xTc[buf]], lin.at[buf], sem_lin)
        return h_rows, h_lin

    handles = fetch(0)
    for c in range(NCHUNK):
        buf = c % NBUF
        h_rows, h_lin = handles
        if c + 1 < NCHUNK:
            handles = fetch(c + 1)
        h_rows.wait()

        def row_body(b, carry, buf=buf):
            v = rows[buf, b, :]
            acc = v
            acc2 = v * v
            for f in range(1, F):
                v = rows[buf, f * CHUNK + b, :]
                acc = acc + v
                acc2 = acc2 + v * v
            tb[pl.ds(b * 16, 16)] = acc * acc - acc2
            return carry

        lax.fori_loop(0, CHUNK, row_body, 0)

        h_lin.wait()

        def grp_body(g, carry, buf=buf, c=c):
            colbase = g * 256 + lax.iota(jnp.int32, 16) * 16
            sv = plsc.load_gather(tb, [colbase])
            for j in range(1, 16):
                sv = sv + plsc.load_gather(tb, [colbase + j])
            lv = lin[buf, pl.ds(g * 16, 16)]
            for f in range(1, F):
                lv = lv + lin[buf, pl.ds(f * CHUNK + g * 16, 16)]
            outb[pl.ds(c * CHUNK + g * 16, 16)] = 0.5 * sv + lv + bv
            return carry

        lax.fori_loop(0, CHUNK // 16, grp_body, 0)

    pltpu.sync_copy(outb, out_hbm.at[pl.ds(base, ROWS_PER_W)])


@jax.jit
def _fm_call(idx, wso, wlin, bias16):
    mesh = plsc.VectorSubcoreMesh(core_axis_name="c", subcore_axis_name="s")
    return pl.kernel(
        _fm_body,
        mesh=mesh,
        compiler_params=pltpu.CompilerParams(
            needs_layout_passes=False, use_tc_tiling_on_sc=False),
        out_type=jax.ShapeDtypeStruct((B,), jnp.float32),
        scratch_types=[
            pltpu.VMEM((F * CHUNK,), jnp.int32),
            pltpu.VMEM((F * CHUNK,), jnp.int32),
            pltpu.VMEM((NBUF, F * CHUNK, K), jnp.float32),
            pltpu.VMEM((NBUF, F * CHUNK), jnp.float32),
            pltpu.VMEM((CHUNK * 16,), jnp.float32),
            pltpu.VMEM((ROWS_PER_W,), jnp.float32),
            pltpu.VMEM((16,), jnp.float32),
            pltpu.SemaphoreType.DMA,
            pltpu.SemaphoreType.DMA,
            pltpu.SemaphoreType.DMA,
        ],
    )(idx, wso, wlin, bias16)


def kernel(sparse_x, W_lin, W_so, bias):
    offsets = jnp.arange(F, dtype=sparse_x.dtype) * FIELD
    x = sparse_x + offsets[None, :]
    idx = (x.reshape(NW, NCHUNK, CHUNK, F)
            .transpose(0, 1, 3, 2)
            .reshape(NW, NCHUNK, F * CHUNK))
    wlin1 = W_lin.T[0]
    wt = W_so.T
    wtail = W_so[TOTAL - 128:].T
    wso_rm = _tr_call(wt, wtail).reshape(TOTAL, K)
    bias16 = jnp.broadcast_to(bias, (16,))
    return _fm_call(idx, wso_rm, wlin1, bias16)


# speedup vs baseline: 1.0040x; 1.0040x over previous
import jax
import jax.numpy as jnp
from jax import lax
from jax.experimental import pallas as pl
from jax.experimental.pallas import tpu as pltpu
from jax.experimental.pallas import tpu_sc as plsc

B = 16384
F = 26
FIELD = 100000
TOTAL = F * FIELD
K = 16

NC = 2
NS = 16
NW = NC * NS
ROWS_PER_W = B // NW
CHUNK = 128
NCHUNK = ROWS_PER_W // CHUNK
NBUF = 2


SB = 1024
NSB = TOTAL // SB
NTAIL = TOTAL - NSB * SB


def _tr_body(wt_hbm, wtail_hbm, out_hbm, xb0, xb1, yb0, yb1,
             si0, si1, so0, so1):
    wid = lax.axis_index("s") * NC + lax.axis_index("c")
    lanes = lax.iota(jnp.int32, 16)
    nfull = NSB // NW + jnp.where(wid < NSB % NW, 1, 0)

    def start_in(b, xb, si):
        pltpu.async_copy(wt_hbm.at[pl.ds(0, 8), pl.ds(b * SB, SB)],
                         xb.at[pl.ds(0, 8), pl.ds(0, SB)], si)
        pltpu.async_copy(wt_hbm.at[pl.ds(8, 8), pl.ds(b * SB, SB)],
                         xb.at[pl.ds(8, 8), pl.ds(0, SB)], si)

    def wait_in(xb, si):
        pltpu.make_async_copy(wt_hbm.at[:, pl.ds(0, SB)],
                              xb.at[:, pl.ds(0, SB)], si).wait()

    def wait_out(yb, so):
        pltpu.make_async_copy(out_hbm.at[pl.ds(0, SB // 8), :], yb, so).wait()

    def permute(xb, yb, j0, nj8):
        def col_grp(j8, jv):
            for jj in range(8):
                col = plsc.load_gather(xb, [lanes, jv])
                yb[j8, pl.ds(jj * 16, 16)] = col
                jv = jv + 1
            return jv

        lax.fori_loop(0, nj8, col_grp, jnp.broadcast_to(j0, (16,)))

    npair = nfull // 2
    odd = nfull - npair * 2

    start_in(wid, xb0, si0)

    def pair_body(i2, carry):
        b0 = wid + (2 * i2) * NW
        start_in(b0 + NW, xb1, si1)

        @pl.when(i2 >= 1)
        def _():
            wait_out(yb0, so0)

        wait_in(xb0, si0)
        permute(xb0, yb0, 0, SB // 8)
        pltpu.async_copy(yb0, out_hbm.at[pl.ds(b0 * (SB // 8), SB // 8), :],
                         so0)

        @pl.when(2 * i2 + 2 < nfull)
        def _():
            start_in(b0 + 2 * NW, xb0, si0)

        @pl.when(i2 >= 1)
        def _():
            wait_out(yb1, so1)

        wait_in(xb1, si1)
        permute(xb1, yb1, 0, SB // 8)
        pltpu.async_copy(yb1,
                         out_hbm.at[pl.ds((b0 + NW) * (SB // 8), SB // 8), :],
                         so1)
        return carry

    @pl.when(npair >= 1)
    def _():
        lax.fori_loop(0, npair, pair_body, 0)
        wait_out(yb0, so0)
        wait_out(yb1, so1)

    @pl.when(odd == 1)
    def _():
        b = wid + (nfull - 1) * NW
        wait_in(xb0, si0)
        permute(xb0, yb0, 0, SB // 8)
        pltpu.async_copy(yb0, out_hbm.at[pl.ds(b * (SB // 8), SB // 8), :],
                         so0).wait()

    @pl.when(wid == NW - 1)
    def _tail():
        pltpu.async_copy(wtail_hbm, xb0.at[:, pl.ds(0, 128)], si0).wait()
        permute(xb0, yb0, 64, NTAIL // 8)
        pltpu.async_copy(yb0.at[pl.ds(0, NTAIL // 8), :],
                         out_hbm.at[pl.ds(NSB * (SB // 8), NTAIL // 8), :],
                         so0).wait()


@jax.jit
def _tr_call(wt, wtail):
    mesh = plsc.VectorSubcoreMesh(core_axis_name="c", subcore_axis_name="s")
    return pl.kernel(
        _tr_body,
        mesh=mesh,
        compiler_params=pltpu.CompilerParams(
            needs_layout_passes=False, use_tc_tiling_on_sc=True),
        out_type=jax.ShapeDtypeStruct((TOTAL * K // 128, 128), jnp.float32),
        scratch_types=[
            pltpu.VMEM((16, SB + 1), jnp.float32),
            pltpu.VMEM((16, SB + 1), jnp.float32),
            pltpu.VMEM((SB // 8, 128), jnp.float32),
            pltpu.VMEM((SB // 8, 128), jnp.float32),
            pltpu.SemaphoreType.DMA,
            pltpu.SemaphoreType.DMA,
            pltpu.SemaphoreType.DMA,
            pltpu.SemaphoreType.DMA,
        ],
    )(wt, wtail)


def _fm_body(idx_hbm, wso_hbm, wlin_hbm, bias_hbm, out_hbm,
             xTc0, xTc1, rows, lin, tb, outb, biasv, sem_idx, sem_rows, sem_lin):
    xTc = (xTc0, xTc1)
    wid = lax.axis_index("s") * NC + lax.axis_index("c")
    base = wid * ROWS_PER_W

    pltpu.sync_copy(bias_hbm, biasv)
    bv = biasv[...]

    def fetch(c):
        buf = c % NBUF
        pltpu.sync_copy(idx_hbm.at[wid, c], xTc[buf])
        h_rows = pltpu.async_copy(wso_hbm.at[xTc[buf]], rows.at[buf], sem_rows)
        h_lin = pltpu.async_copy(wlin_hbm.at[xTc[buf]], lin.at[buf], sem_lin)
        return h_rows, h_lin

    handles = fetch(0)
    for c in range(NCHUNK):
        buf = c % NBUF
        h_rows, h_lin = handles
        if c + 1 < NCHUNK:
            handles = fetch(c + 1)
        h_rows.wait()

        def row_body(b, carry, buf=buf):
            v = rows[buf, b, :]
            acc = v
            acc2 = v * v
            for f in range(1, F):
                v = rows[buf, f * CHUNK + b, :]
                acc = acc + v
                acc2 = acc2 + v * v
            tb[pl.ds(b * 16, 16)] = acc * acc - acc2
            return carry

        lax.fori_loop(0, CHUNK, row_body, 0)

        h_lin.wait()

        def grp_body(g, carry, buf=buf, c=c):
            colbase = g * 256 + lax.iota(jnp.int32, 16) * 16
            sv = plsc.load_gather(tb, [colbase])
            for j in range(1, 16):
                sv = sv + plsc.load_gather(tb, [colbase + j])
            lv = lin[buf, pl.ds(g * 16, 16)]
            for f in range(1, F):
                lv = lv + lin[buf, pl.ds(f * CHUNK + g * 16, 16)]
            outb[pl.ds(c * CHUNK + g * 16, 16)] = 0.5 * sv + lv + bv
            return carry

        lax.fori_loop(0, CHUNK // 16, grp_body, 0)

    pltpu.sync_copy(outb, out_hbm.at[pl.ds(base, ROWS_PER_W)])


@jax.jit
def _fm_call(idx, wso, wlin, bias16):
    mesh = plsc.VectorSubcoreMesh(core_axis_name="c", subcore_axis_name="s")
    return pl.kernel(
        _fm_body,
        mesh=mesh,
        compiler_params=pltpu.CompilerParams(
            needs_layout_passes=False, use_tc_tiling_on_sc=False),
        out_type=jax.ShapeDtypeStruct((B,), jnp.float32),
        scratch_types=[
            pltpu.VMEM((F * CHUNK,), jnp.int32),
            pltpu.VMEM((F * CHUNK,), jnp.int32),
            pltpu.VMEM((NBUF, F * CHUNK, K), jnp.float32),
            pltpu.VMEM((NBUF, F * CHUNK), jnp.float32),
            pltpu.VMEM((CHUNK * 16,), jnp.float32),
            pltpu.VMEM((ROWS_PER_W,), jnp.float32),
            pltpu.VMEM((16,), jnp.float32),
            pltpu.SemaphoreType.DMA,
            pltpu.SemaphoreType.DMA,
            pltpu.SemaphoreType.DMA,
        ],
    )(idx, wso, wlin, bias16)


def kernel(sparse_x, W_lin, W_so, bias):
    offsets = jnp.arange(F, dtype=sparse_x.dtype) * FIELD
    x = sparse_x + offsets[None, :]
    idx = (x.reshape(NW, NCHUNK, CHUNK, F)
            .transpose(0, 1, 3, 2)
            .reshape(NW, NCHUNK, F * CHUNK))
    wlin1 = W_lin.T[0]
    wt = W_so.T
    wtail = W_so[TOTAL - 128:].T
    wso_rm = _tr_call(wt, wtail).reshape(TOTAL, K)
    bias16 = jnp.broadcast_to(bias, (16,))
    return _fm_call(idx, wso_rm, wlin1, bias16)


# speedup vs baseline: 1.6343x; 1.6278x over previous
import jax
import jax.numpy as jnp
from jax import lax
from jax.experimental import pallas as pl
from jax.experimental.pallas import tpu as pltpu
from jax.experimental.pallas import tpu_sc as plsc

B = 16384
F = 26
FIELD = 100000
TOTAL = F * FIELD
K = 16

NC = 2
NS = 16
NW = NC * NS
ROWS_PER_W = B // NW
CHUNK = 128
NCHUNK = ROWS_PER_W // CHUNK
NBUF = 2


SB = 1024
NSB = TOTAL // SB
NTAIL = TOTAL - NSB * SB


def _tr_body(wt_hbm, wtail_hbm, out_hbm, xb0, xb1, yb0, yb1,
             si0, si1, so0, so1):
    wid = lax.axis_index("s") * NC + lax.axis_index("c")
    lanes = lax.iota(jnp.int32, 16)
    nfull = NSB // NW + jnp.where(wid < NSB % NW, 1, 0)

    def start_in(b, xb, si):
        pltpu.async_copy(wt_hbm.at[pl.ds(0, 8), pl.ds(b * SB, SB)],
                         xb.at[pl.ds(0, 8), pl.ds(0, SB)], si)
        pltpu.async_copy(wt_hbm.at[pl.ds(8, 8), pl.ds(b * SB, SB)],
                         xb.at[pl.ds(8, 8), pl.ds(0, SB)], si)

    def wait_in(xb, si):
        pltpu.make_async_copy(wt_hbm.at[:, pl.ds(0, SB)],
                              xb.at[:, pl.ds(0, SB)], si).wait()

    def wait_out(yb, so):
        pltpu.make_async_copy(out_hbm.at[pl.ds(0, SB // 8), :], yb, so).wait()

    def permute(xb, yb, j0, nj8):
        def col_grp(j8, jv):
            cols = [plsc.load_gather(xb, [lanes, jv + jj]) for jj in range(8)]
            for jj in range(8):
                yb[j8, pl.ds(jj * 16, 16)] = cols[jj]
            return jv + 8

        lax.fori_loop(0, nj8, col_grp, jnp.broadcast_to(j0, (16,)))

    npair = nfull // 2
    odd = nfull - npair * 2

    start_in(wid, xb0, si0)

    def pair_body(i2, carry):
        b0 = wid + (2 * i2) * NW
        start_in(b0 + NW, xb1, si1)

        @pl.when(i2 >= 1)
        def _():
            wait_out(yb0, so0)

        wait_in(xb0, si0)
        permute(xb0, yb0, 0, SB // 8)
        pltpu.async_copy(yb0, out_hbm.at[pl.ds(b0 * (SB // 8), SB // 8), :],
                         so0)

        @pl.when(2 * i2 + 2 < nfull)
        def _():
            start_in(b0 + 2 * NW, xb0, si0)

        @pl.when(i2 >= 1)
        def _():
            wait_out(yb1, so1)

        wait_in(xb1, si1)
        permute(xb1, yb1, 0, SB // 8)
        pltpu.async_copy(yb1,
                         out_hbm.at[pl.ds((b0 + NW) * (SB // 8), SB // 8), :],
                         so1)
        return carry

    @pl.when(npair >= 1)
    def _():
        lax.fori_loop(0, npair, pair_body, 0)
        wait_out(yb0, so0)
        wait_out(yb1, so1)

    @pl.when(odd == 1)
    def _():
        b = wid + (nfull - 1) * NW
        wait_in(xb0, si0)
        permute(xb0, yb0, 0, SB // 8)
        pltpu.async_copy(yb0, out_hbm.at[pl.ds(b * (SB // 8), SB // 8), :],
                         so0).wait()

    @pl.when(wid == NW - 1)
    def _tail():
        pltpu.async_copy(wtail_hbm, xb0.at[:, pl.ds(0, 128)], si0).wait()
        permute(xb0, yb0, 64, NTAIL // 8)
        pltpu.async_copy(yb0.at[pl.ds(0, NTAIL // 8), :],
                         out_hbm.at[pl.ds(NSB * (SB // 8), NTAIL // 8), :],
                         so0).wait()


@jax.jit
def _tr_call(wt, wtail):
    mesh = plsc.VectorSubcoreMesh(core_axis_name="c", subcore_axis_name="s")
    return pl.kernel(
        _tr_body,
        mesh=mesh,
        compiler_params=pltpu.CompilerParams(
            needs_layout_passes=False, use_tc_tiling_on_sc=True),
        out_type=jax.ShapeDtypeStruct((TOTAL * K // 128, 128), jnp.float32),
        scratch_types=[
            pltpu.VMEM((16, SB + 1), jnp.float32),
            pltpu.VMEM((16, SB + 1), jnp.float32),
            pltpu.VMEM((SB // 8, 128), jnp.float32),
            pltpu.VMEM((SB // 8, 128), jnp.float32),
            pltpu.SemaphoreType.DMA,
            pltpu.SemaphoreType.DMA,
            pltpu.SemaphoreType.DMA,
            pltpu.SemaphoreType.DMA,
        ],
    )(wt, wtail)


def _fm_body(idx_hbm, wso_hbm, wlin_hbm, bias_hbm, out_hbm,
             xTc0, xTc1, rows, lin, tb, outb, biasv, sem_idx, sem_rows, sem_lin):
    xTc = (xTc0, xTc1)
    wid = lax.axis_index("s") * NC + lax.axis_index("c")
    base = wid * ROWS_PER_W

    pltpu.sync_copy(bias_hbm, biasv)
    bv = biasv[...]

    def fetch(c):
        buf = c % NBUF
        pltpu.sync_copy(idx_hbm.at[wid, c], xTc[buf])
        h_rows = pltpu.async_copy(wso_hbm.at[xTc[buf]], rows.at[buf], sem_rows)
        h_lin = pltpu.async_copy(wlin_hbm.at[xTc[buf]], lin.at[buf], sem_lin)
        return h_rows, h_lin

    handles = fetch(0)
    for c in range(NCHUNK):
        buf = c % NBUF
        h_rows, h_lin = handles
        if c + 1 < NCHUNK:
            handles = fetch(c + 1)
        h_rows.wait()

        def row_body(b, carry, buf=buf):
            v = rows[buf, b, :]
            acc = v
            acc2 = v * v
            for f in range(1, F):
                v = rows[buf, f * CHUNK + b, :]
                acc = acc + v
                acc2 = acc2 + v * v
            tb[pl.ds(b * 16, 16)] = acc * acc - acc2
            return carry

        lax.fori_loop(0, CHUNK, row_body, 0)

        h_lin.wait()

        def grp_body(g, carry, buf=buf, c=c):
            colbase = g * 256 + lax.iota(jnp.int32, 16) * 16
            sv = plsc.load_gather(tb, [colbase])
            for j in range(1, 16):
                sv = sv + plsc.load_gather(tb, [colbase + j])
            lv = lin[buf, pl.ds(g * 16, 16)]
            for f in range(1, F):
                lv = lv + lin[buf, pl.ds(f * CHUNK + g * 16, 16)]
            outb[pl.ds(c * CHUNK + g * 16, 16)] = 0.5 * sv + lv + bv
            return carry

        lax.fori_loop(0, CHUNK // 16, grp_body, 0)

    pltpu.sync_copy(outb, out_hbm.at[pl.ds(base, ROWS_PER_W)])


@jax.jit
def _fm_call(idx, wso, wlin, bias16):
    mesh = plsc.VectorSubcoreMesh(core_axis_name="c", subcore_axis_name="s")
    return pl.kernel(
        _fm_body,
        mesh=mesh,
        compiler_params=pltpu.CompilerParams(
            needs_layout_passes=False, use_tc_tiling_on_sc=False),
        out_type=jax.ShapeDtypeStruct((B,), jnp.float32),
        scratch_types=[
            pltpu.VMEM((F * CHUNK,), jnp.int32),
            pltpu.VMEM((F * CHUNK,), jnp.int32),
            pltpu.VMEM((NBUF, F * CHUNK, K), jnp.float32),
            pltpu.VMEM((NBUF, F * CHUNK), jnp.float32),
            pltpu.VMEM((CHUNK * 16,), jnp.float32),
            pltpu.VMEM((ROWS_PER_W,), jnp.float32),
            pltpu.VMEM((16,), jnp.float32),
            pltpu.SemaphoreType.DMA,
            pltpu.SemaphoreType.DMA,
            pltpu.SemaphoreType.DMA,
        ],
    )(idx, wso, wlin, bias16)


def kernel(sparse_x, W_lin, W_so, bias):
    offsets = jnp.arange(F, dtype=sparse_x.dtype) * FIELD
    x = sparse_x + offsets[None, :]
    idx = (x.reshape(NW, NCHUNK, CHUNK, F)
            .transpose(0, 1, 3, 2)
            .reshape(NW, NCHUNK, F * CHUNK))
    wlin1 = W_lin.T[0]
    wt = W_so.T
    wtail = W_so[TOTAL - 128:].T
    wso_rm = _tr_call(wt, wtail).reshape(TOTAL, K)
    bias16 = jnp.broadcast_to(bias, (16,))
    return _fm_call(idx, wso_rm, wlin1, bias16)


# speedup vs baseline: 1.6825x; 1.0295x over previous
import jax
import jax.numpy as jnp
from jax import lax
from jax.experimental import pallas as pl
from jax.experimental.pallas import tpu as pltpu
from jax.experimental.pallas import tpu_sc as plsc

B = 16384
F = 26
FIELD = 100000
TOTAL = F * FIELD
K = 16

NC = 2
NS = 16
NW = NC * NS
ROWS_PER_W = B // NW
CHUNK = 128
NCHUNK = ROWS_PER_W // CHUNK
NBUF = 2


SB = 1024
NSB = TOTAL // SB
NTAIL = TOTAL - NSB * SB


def _tr_body(wt_hbm, wtail_hbm, out_hbm, xb0, xb1, yb0, yb1,
             si0, si1, so0, so1):
    wid = lax.axis_index("s") * NC + lax.axis_index("c")
    lanes = lax.iota(jnp.int32, 16)
    nfull = NSB // NW + jnp.where(wid < NSB % NW, 1, 0)

    def start_in(b, xb, si):
        pltpu.async_copy(wt_hbm.at[pl.ds(0, 8), pl.ds(b * SB, SB)],
                         xb.at[pl.ds(0, 8), pl.ds(0, SB)], si)
        pltpu.async_copy(wt_hbm.at[pl.ds(8, 8), pl.ds(b * SB, SB)],
                         xb.at[pl.ds(8, 8), pl.ds(0, SB)], si)

    def wait_in(xb, si):
        pltpu.make_async_copy(wt_hbm.at[:, pl.ds(0, SB)],
                              xb.at[:, pl.ds(0, SB)], si).wait()

    def wait_out(yb, so):
        pltpu.make_async_copy(out_hbm.at[pl.ds(0, SB // 8), :], yb, so).wait()

    def permute(xb, yb, j0, nj8):
        def col_grp(g, jv):
            cols = [plsc.load_gather(xb, [lanes, jv + jj]) for jj in range(32)]
            for jj in range(32):
                yb[g * 4 + jj // 8, pl.ds((jj % 8) * 16, 16)] = cols[jj]
            return jv + 32

        lax.fori_loop(0, nj8 // 4, col_grp, jnp.broadcast_to(j0, (16,)))

    npair = nfull // 2
    odd = nfull - npair * 2

    start_in(wid, xb0, si0)

    def pair_body(i2, carry):
        b0 = wid + (2 * i2) * NW
        start_in(b0 + NW, xb1, si1)

        @pl.when(i2 >= 1)
        def _():
            wait_out(yb0, so0)

        wait_in(xb0, si0)
        permute(xb0, yb0, 0, SB // 8)
        pltpu.async_copy(yb0, out_hbm.at[pl.ds(b0 * (SB // 8), SB // 8), :],
                         so0)

        @pl.when(2 * i2 + 2 < nfull)
        def _():
            start_in(b0 + 2 * NW, xb0, si0)

        @pl.when(i2 >= 1)
        def _():
            wait_out(yb1, so1)

        wait_in(xb1, si1)
        permute(xb1, yb1, 0, SB // 8)
        pltpu.async_copy(yb1,
                         out_hbm.at[pl.ds((b0 + NW) * (SB // 8), SB // 8), :],
                         so1)
        return carry

    @pl.when(npair >= 1)
    def _():
        lax.fori_loop(0, npair, pair_body, 0)
        wait_out(yb0, so0)
        wait_out(yb1, so1)

    @pl.when(odd == 1)
    def _():
        b = wid + (nfull - 1) * NW
        wait_in(xb0, si0)
        permute(xb0, yb0, 0, SB // 8)
        pltpu.async_copy(yb0, out_hbm.at[pl.ds(b * (SB // 8), SB // 8), :],
                         so0).wait()

    @pl.when(wid == NW - 1)
    def _tail():
        pltpu.async_copy(wtail_hbm, xb0.at[:, pl.ds(0, 128)], si0).wait()
        permute(xb0, yb0, 64, NTAIL // 8)
        pltpu.async_copy(yb0.at[pl.ds(0, NTAIL // 8), :],
                         out_hbm.at[pl.ds(NSB * (SB // 8), NTAIL // 8), :],
                         so0).wait()


@jax.jit
def _tr_call(wt, wtail):
    mesh = plsc.VectorSubcoreMesh(core_axis_name="c", subcore_axis_name="s")
    return pl.kernel(
        _tr_body,
        mesh=mesh,
        compiler_params=pltpu.CompilerParams(
            needs_layout_passes=False, use_tc_tiling_on_sc=True),
        out_type=jax.ShapeDtypeStruct((TOTAL * K // 128, 128), jnp.float32),
        scratch_types=[
            pltpu.VMEM((16, SB + 1), jnp.float32),
            pltpu.VMEM((16, SB + 1), jnp.float32),
            pltpu.VMEM((SB // 8, 128), jnp.float32),
            pltpu.VMEM((SB // 8, 128), jnp.float32),
            pltpu.SemaphoreType.DMA,
            pltpu.SemaphoreType.DMA,
            pltpu.SemaphoreType.DMA,
            pltpu.SemaphoreType.DMA,
        ],
    )(wt, wtail)


def _fm_body(idx_hbm, wso_hbm, wlin_hbm, bias_hbm, out_hbm,
             xTc0, xTc1, rows, lin, tb, outb, biasv, sem_idx, sem_rows, sem_lin):
    xTc = (xTc0, xTc1)
    wid = lax.axis_index("s") * NC + lax.axis_index("c")
    base = wid * ROWS_PER_W

    pltpu.sync_copy(bias_hbm, biasv)
    bv = biasv[...]

    def fetch(c):
        buf = c % NBUF
        pltpu.sync_copy(idx_hbm.at[wid, c], xTc[buf])
        h_rows = pltpu.async_copy(wso_hbm.at[xTc[buf]], rows.at[buf], sem_rows)
        h_lin = pltpu.async_copy(wlin_hbm.at[xTc[buf]], lin.at[buf], sem_lin)
        return h_rows, h_lin

    handles = fetch(0)
    for c in range(NCHUNK):
        buf = c % NBUF
        h_rows, h_lin = handles
        if c + 1 < NCHUNK:
            handles = fetch(c + 1)
        h_rows.wait()

        def row_body(b, carry, buf=buf):
            v = rows[buf, b, :]
            acc = v
            acc2 = v * v
            for f in range(1, F):
                v = rows[buf, f * CHUNK + b, :]
                acc = acc + v
                acc2 = acc2 + v * v
            tb[pl.ds(b * 16, 16)] = acc * acc - acc2
            return carry

        lax.fori_loop(0, CHUNK, row_body, 0)

        h_lin.wait()

        def grp_body(g, carry, buf=buf, c=c):
            colbase = g * 256 + lax.iota(jnp.int32, 16) * 16
            sv = plsc.load_gather(tb, [colbase])
            for j in range(1, 16):
                sv = sv + plsc.load_gather(tb, [colbase + j])
            lv = lin[buf, pl.ds(g * 16, 16)]
            for f in range(1, F):
                lv = lv + lin[buf, pl.ds(f * CHUNK + g * 16, 16)]
            outb[pl.ds(c * CHUNK + g * 16, 16)] = 0.5 * sv + lv + bv
            return carry

        lax.fori_loop(0, CHUNK // 16, grp_body, 0)

    pltpu.sync_copy(outb, out_hbm.at[pl.ds(base, ROWS_PER_W)])


@jax.jit
def _fm_call(idx, wso, wlin, bias16):
    mesh = plsc.VectorSubcoreMesh(core_axis_name="c", subcore_axis_name="s")
    return pl.kernel(
        _fm_body,
        mesh=mesh,
        compiler_params=pltpu.CompilerParams(
            needs_layout_passes=False, use_tc_tiling_on_sc=False),
        out_type=jax.ShapeDtypeStruct((B,), jnp.float32),
        scratch_types=[
            pltpu.VMEM((F * CHUNK,), jnp.int32),
            pltpu.VMEM((F * CHUNK,), jnp.int32),
            pltpu.VMEM((NBUF, F * CHUNK, K), jnp.float32),
            pltpu.VMEM((NBUF, F * CHUNK), jnp.float32),
            pltpu.VMEM((CHUNK * 16,), jnp.float32),
            pltpu.VMEM((ROWS_PER_W,), jnp.float32),
            pltpu.VMEM((16,), jnp.float32),
            pltpu.SemaphoreType.DMA,
            pltpu.SemaphoreType.DMA,
            pltpu.SemaphoreType.DMA,
        ],
    )(idx, wso, wlin, bias16)


def kernel(sparse_x, W_lin, W_so, bias):
    offsets = jnp.arange(F, dtype=sparse_x.dtype) * FIELD
    x = sparse_x + offsets[None, :]
    idx = (x.reshape(NW, NCHUNK, CHUNK, F)
            .transpose(0, 1, 3, 2)
            .reshape(NW, NCHUNK, F * CHUNK))
    wlin1 = W_lin.T[0]
    wt = W_so.T
    wtail = W_so[TOTAL - 128:].T
    wso_rm = _tr_call(wt, wtail).reshape(TOTAL, K)
    bias16 = jnp.broadcast_to(bias, (16,))
    return _fm_call(idx, wso_rm, wlin1, bias16)


# speedup vs baseline: 2.6189x; 1.5565x over previous
import jax
import jax.numpy as jnp
from jax import lax
from jax.experimental import pallas as pl
from jax.experimental.pallas import tpu as pltpu
from jax.experimental.pallas import tpu_sc as plsc

B = 16384
F = 26
FIELD = 100000
TOTAL = F * FIELD
K = 16

NC = 2
NS = 16
NW = NC * NS
ROWS_PER_W = B // NW
CHUNK = 128
NCHUNK = ROWS_PER_W // CHUNK
NBUF = 2


SB = 1024
NSB = TOTAL // SB
NTAIL = TOTAL - NSB * SB


def _tr_body(wt_hbm, wtail_hbm, out_hbm, xb0, xb1, zb0, zb1,
             si0, si1, so0, so1):
    wid = lax.axis_index("s") * NC + lax.axis_index("c")
    lanes = lax.iota(jnp.int32, 16)
    nfull = NSB // NW + jnp.where(wid < NSB % NW, 1, 0)

    def start_in(b, xb, si):
        pltpu.async_copy(wt_hbm.at[pl.ds(0, 8), pl.ds(b * SB, SB)],
                         xb.at[pl.ds(0, 8), :], si)
        pltpu.async_copy(wt_hbm.at[pl.ds(8, 8), pl.ds(b * SB, SB)],
                         xb.at[pl.ds(8, 8), :], si)

    def wait_in(xb, si):
        pltpu.make_async_copy(wt_hbm.at[:, pl.ds(0, SB)], xb, si).wait()

    def wait_out(zb, so):
        pltpu.make_async_copy(out_hbm.at[pl.ds(0, SB * K)], zb, so).wait()

    def permute(xb, zb, ncol, cbase):
        mask = ncol - 1

        def col_grp(g, t):
            for jj in range(16):
                cv = cbase + ((t + jj) & mask)
                col = plsc.load_gather(xb, [lanes, cv])
                sv = (cv - cbase) * 16 + lanes
                plsc.store_scatter(zb, [sv], col)
            return t + 16

        lax.fori_loop(0, ncol // 16, col_grp, lanes * 17)

    npair = nfull // 2
    odd = nfull - npair * 2

    start_in(wid, xb0, si0)

    def pair_body(i2, carry):
        b0 = wid + (2 * i2) * NW
        start_in(b0 + NW, xb1, si1)

        @pl.when(i2 >= 1)
        def _():
            wait_out(zb0, so0)

        wait_in(xb0, si0)
        permute(xb0, zb0, SB, 0)
        pltpu.async_copy(zb0, out_hbm.at[pl.ds(b0 * (SB * K), SB * K)], so0)

        @pl.when(2 * i2 + 2 < nfull)
        def _():
            start_in(b0 + 2 * NW, xb0, si0)

        @pl.when(i2 >= 1)
        def _():
            wait_out(zb1, so1)

        wait_in(xb1, si1)
        permute(xb1, zb1, SB, 0)
        pltpu.async_copy(zb1, out_hbm.at[pl.ds((b0 + NW) * (SB * K), SB * K)],
                         so1)
        return carry

    @pl.when(npair >= 1)
    def _():
        lax.fori_loop(0, npair, pair_body, 0)
        wait_out(zb0, so0)
        wait_out(zb1, so1)

    @pl.when(odd == 1)
    def _():
        b = wid + (nfull - 1) * NW
        wait_in(xb0, si0)
        permute(xb0, zb0, SB, 0)
        pltpu.async_copy(zb0, out_hbm.at[pl.ds(b * (SB * K), SB * K)],
                         so0).wait()

    @pl.when(wid == NW - 1)
    def _tail():
        pltpu.async_copy(wtail_hbm, xb0.at[:, pl.ds(0, 128)], si0).wait()
        permute(xb0, zb0, NTAIL, 64)
        pltpu.async_copy(zb0.at[pl.ds(0, NTAIL * K)],
                         out_hbm.at[pl.ds(NSB * SB * K, NTAIL * K)],
                         so0).wait()


@jax.jit
def _tr_call(wt, wtail):
    mesh = plsc.VectorSubcoreMesh(core_axis_name="c", subcore_axis_name="s")
    return pl.kernel(
        _tr_body,
        mesh=mesh,
        compiler_params=pltpu.CompilerParams(
            needs_layout_passes=False, use_tc_tiling_on_sc=True),
        out_type=jax.ShapeDtypeStruct((TOTAL * K,), jnp.float32),
        scratch_types=[
            pltpu.VMEM((16, SB), jnp.float32),
            pltpu.VMEM((16, SB), jnp.float32),
            pltpu.VMEM((SB * K,), jnp.float32),
            pltpu.VMEM((SB * K,), jnp.float32),
            pltpu.SemaphoreType.DMA,
            pltpu.SemaphoreType.DMA,
            pltpu.SemaphoreType.DMA,
            pltpu.SemaphoreType.DMA,
        ],
    )(wt, wtail)


def _fm_body(idx_hbm, wso_hbm, wlin_hbm, bias_hbm, out_hbm,
             xTc0, xTc1, rows, lin, tb, outb, biasv, sem_idx, sem_rows, sem_lin):
    xTc = (xTc0, xTc1)
    wid = lax.axis_index("s") * NC + lax.axis_index("c")
    base = wid * ROWS_PER_W

    pltpu.sync_copy(bias_hbm, biasv)
    bv = biasv[...]

    def fetch(c):
        buf = c % NBUF
        pltpu.sync_copy(idx_hbm.at[wid, c], xTc[buf])
        h_rows = pltpu.async_copy(wso_hbm.at[xTc[buf]], rows.at[buf], sem_rows)
        h_lin = pltpu.async_copy(wlin_hbm.at[xTc[buf]], lin.at[buf], sem_lin)
        return h_rows, h_lin

    handles = fetch(0)
    for c in range(NCHUNK):
        buf = c % NBUF
        h_rows, h_lin = handles
        if c + 1 < NCHUNK:
            handles = fetch(c + 1)
        h_rows.wait()

        def row_body(b, carry, buf=buf):
            v = rows[buf, b, :]
            acc = v
            acc2 = v * v
            for f in range(1, F):
                v = rows[buf, f * CHUNK + b, :]
                acc = acc + v
                acc2 = acc2 + v * v
            tb[pl.ds(b * 16, 16)] = acc * acc - acc2
            return carry

        lax.fori_loop(0, CHUNK, row_body, 0)

        h_lin.wait()

        def grp_body(g, carry, buf=buf, c=c):
            colbase = g * 256 + lax.iota(jnp.int32, 16) * 16
            sv = plsc.load_gather(tb, [colbase])
            for j in range(1, 16):
                sv = sv + plsc.load_gather(tb, [colbase + j])
            lv = lin[buf, pl.ds(g * 16, 16)]
            for f in range(1, F):
                lv = lv + lin[buf, pl.ds(f * CHUNK + g * 16, 16)]
            outb[pl.ds(c * CHUNK + g * 16, 16)] = 0.5 * sv + lv + bv
            return carry

        lax.fori_loop(0, CHUNK // 16, grp_body, 0)

    pltpu.sync_copy(outb, out_hbm.at[pl.ds(base, ROWS_PER_W)])


@jax.jit
def _fm_call(idx, wso, wlin, bias16):
    mesh = plsc.VectorSubcoreMesh(core_axis_name="c", subcore_axis_name="s")
    return pl.kernel(
        _fm_body,
        mesh=mesh,
        compiler_params=pltpu.CompilerParams(
            needs_layout_passes=False, use_tc_tiling_on_sc=False),
        out_type=jax.ShapeDtypeStruct((B,), jnp.float32),
        scratch_types=[
            pltpu.VMEM((F * CHUNK,), jnp.int32),
            pltpu.VMEM((F * CHUNK,), jnp.int32),
            pltpu.VMEM((NBUF, F * CHUNK, K), jnp.float32),
            pltpu.VMEM((NBUF, F * CHUNK), jnp.float32),
            pltpu.VMEM((CHUNK * 16,), jnp.float32),
            pltpu.VMEM((ROWS_PER_W,), jnp.float32),
            pltpu.VMEM((16,), jnp.float32),
            pltpu.SemaphoreType.DMA,
            pltpu.SemaphoreType.DMA,
            pltpu.SemaphoreType.DMA,
        ],
    )(idx, wso, wlin, bias16)


def kernel(sparse_x, W_lin, W_so, bias):
    offsets = jnp.arange(F, dtype=sparse_x.dtype) * FIELD
    x = sparse_x + offsets[None, :]
    idx = (x.reshape(NW, NCHUNK, CHUNK, F)
            .transpose(0, 1, 3, 2)
            .reshape(NW, NCHUNK, F * CHUNK))
    wlin1 = W_lin.T[0]
    wt = W_so.T
    wtail = W_so[TOTAL - 128:].T
    wso_rm = _tr_call(wt, wtail).reshape(TOTAL, K)
    bias16 = jnp.broadcast_to(bias, (16,))
    return _fm_call(idx, wso_rm, wlin1, bias16)


# speedup vs baseline: 4.9503x; 1.8902x over previous
import jax
import jax.numpy as jnp
from jax import lax
from jax.experimental import pallas as pl
from jax.experimental.pallas import tpu as pltpu
from jax.experimental.pallas import tpu_sc as plsc

B = 16384
F = 26
FIELD = 100000
TOTAL = F * FIELD
K = 16

NC = 2
NS = 16
NW = NC * NS
ROWS_PER_W = B // NW
CHUNK = 128
NCHUNK = ROWS_PER_W // CHUNK
NBUF = 2


SB = 1024
NSB = TOTAL // SB
NTAIL = TOTAL - NSB * SB


def _tr_body(wt_hbm, wtail_hbm, out_hbm, xb0, xb1, zb0, zb1,
             si0, si1, so0, so1):
    wid = lax.axis_index("s") * NC + lax.axis_index("c")
    lanes = lax.iota(jnp.int32, 16)
    nfull = NSB // NW + jnp.where(wid < NSB % NW, 1, 0)

    def start_in(b, xb, si):
        pltpu.async_copy(wt_hbm.at[pl.ds(0, 8), pl.ds(b * SB, SB)],
                         xb.at[pl.ds(0, 8), :], si)
        pltpu.async_copy(wt_hbm.at[pl.ds(8, 8), pl.ds(b * SB, SB)],
                         xb.at[pl.ds(8, 8), :], si)

    def wait_in(xb, si):
        pltpu.make_async_copy(wt_hbm.at[:, pl.ds(0, SB)], xb, si).wait()

    def wait_out(zb, so):
        pltpu.make_async_copy(out_hbm.at[pl.ds(0, SB * K)], zb, so).wait()

    def permute(xb, zb, ncol, cbase):
        mask = ncol - 1
        lanes17 = lanes * 17

        @plsc.parallel_loop(0, ncol // 16, step=1, unroll=2)
        def _(g):
            t = lanes17 + g * 16
            for jj in range(16):
                cv = cbase + ((t + jj) & mask)
                col = plsc.load_gather(xb, [lanes, cv])
                sv = (cv - cbase) * 16 + lanes
                plsc.store_scatter(zb, [sv], col)

    npair = nfull // 2
    odd = nfull - npair * 2

    start_in(wid, xb0, si0)

    def pair_body(i2, carry):
        b0 = wid + (2 * i2) * NW
        start_in(b0 + NW, xb1, si1)

        @pl.when(i2 >= 1)
        def _():
            wait_out(zb0, so0)

        wait_in(xb0, si0)
        permute(xb0, zb0, SB, 0)
        pltpu.async_copy(zb0, out_hbm.at[pl.ds(b0 * (SB * K), SB * K)], so0)

        @pl.when(2 * i2 + 2 < nfull)
        def _():
            start_in(b0 + 2 * NW, xb0, si0)

        @pl.when(i2 >= 1)
        def _():
            wait_out(zb1, so1)

        wait_in(xb1, si1)
        permute(xb1, zb1, SB, 0)
        pltpu.async_copy(zb1, out_hbm.at[pl.ds((b0 + NW) * (SB * K), SB * K)],
                         so1)
        return carry

    @pl.when(npair >= 1)
    def _():
        lax.fori_loop(0, npair, pair_body, 0)
        wait_out(zb0, so0)
        wait_out(zb1, so1)

    @pl.when(odd == 1)
    def _():
        b = wid + (nfull - 1) * NW
        wait_in(xb0, si0)
        permute(xb0, zb0, SB, 0)
        pltpu.async_copy(zb0, out_hbm.at[pl.ds(b * (SB * K), SB * K)],
                         so0).wait()

    @pl.when(wid == NW - 1)
    def _tail():
        pltpu.async_copy(wtail_hbm, xb0.at[:, pl.ds(0, 128)], si0).wait()
        permute(xb0, zb0, NTAIL, 64)
        pltpu.async_copy(zb0.at[pl.ds(0, NTAIL * K)],
                         out_hbm.at[pl.ds(NSB * SB * K, NTAIL * K)],
                         so0).wait()


@jax.jit
def _tr_call(wt, wtail):
    mesh = plsc.VectorSubcoreMesh(core_axis_name="c", subcore_axis_name="s")
    return pl.kernel(
        _tr_body,
        mesh=mesh,
        compiler_params=pltpu.CompilerParams(
            needs_layout_passes=False, use_tc_tiling_on_sc=True),
        out_type=jax.ShapeDtypeStruct((TOTAL * K,), jnp.float32),
        scratch_types=[
            pltpu.VMEM((16, SB), jnp.float32),
            pltpu.VMEM((16, SB), jnp.float32),
            pltpu.VMEM((SB * K,), jnp.float32),
            pltpu.VMEM((SB * K,), jnp.float32),
            pltpu.SemaphoreType.DMA,
            pltpu.SemaphoreType.DMA,
            pltpu.SemaphoreType.DMA,
            pltpu.SemaphoreType.DMA,
        ],
    )(wt, wtail)


def _fm_body(idx_hbm, wso_hbm, wlin_hbm, bias_hbm, out_hbm,
             xTc0, xTc1, rows, lin, tb, outb, biasv, sem_idx, sem_rows, sem_lin):
    xTc = (xTc0, xTc1)
    wid = lax.axis_index("s") * NC + lax.axis_index("c")
    base = wid * ROWS_PER_W

    pltpu.sync_copy(bias_hbm, biasv)
    bv = biasv[...]

    def fetch(c):
        buf = c % NBUF
        pltpu.sync_copy(idx_hbm.at[wid, c], xTc[buf])
        h_rows = pltpu.async_copy(wso_hbm.at[xTc[buf]], rows.at[buf], sem_rows)
        h_lin = pltpu.async_copy(wlin_hbm.at[xTc[buf]], lin.at[buf], sem_lin)
        return h_rows, h_lin

    handles = fetch(0)
    for c in range(NCHUNK):
        buf = c % NBUF
        h_rows, h_lin = handles
        if c + 1 < NCHUNK:
            handles = fetch(c + 1)
        h_rows.wait()

        def row_body(b, carry, buf=buf):
            v = rows[buf, b, :]
            acc = v
            acc2 = v * v
            for f in range(1, F):
                v = rows[buf, f * CHUNK + b, :]
                acc = acc + v
                acc2 = acc2 + v * v
            tb[pl.ds(b * 16, 16)] = acc * acc - acc2
            return carry

        lax.fori_loop(0, CHUNK, row_body, 0)

        h_lin.wait()

        def grp_body(g, carry, buf=buf, c=c):
            colbase = g * 256 + lax.iota(jnp.int32, 16) * 16
            sv = plsc.load_gather(tb, [colbase])
            for j in range(1, 16):
                sv = sv + plsc.load_gather(tb, [colbase + j])
            lv = lin[buf, pl.ds(g * 16, 16)]
            for f in range(1, F):
                lv = lv + lin[buf, pl.ds(f * CHUNK + g * 16, 16)]
            outb[pl.ds(c * CHUNK + g * 16, 16)] = 0.5 * sv + lv + bv
            return carry

        lax.fori_loop(0, CHUNK // 16, grp_body, 0)

    pltpu.sync_copy(outb, out_hbm.at[pl.ds(base, ROWS_PER_W)])


@jax.jit
def _fm_call(idx, wso, wlin, bias16):
    mesh = plsc.VectorSubcoreMesh(core_axis_name="c", subcore_axis_name="s")
    return pl.kernel(
        _fm_body,
        mesh=mesh,
        compiler_params=pltpu.CompilerParams(
            needs_layout_passes=False, use_tc_tiling_on_sc=False),
        out_type=jax.ShapeDtypeStruct((B,), jnp.float32),
        scratch_types=[
            pltpu.VMEM((F * CHUNK,), jnp.int32),
            pltpu.VMEM((F * CHUNK,), jnp.int32),
            pltpu.VMEM((NBUF, F * CHUNK, K), jnp.float32),
            pltpu.VMEM((NBUF, F * CHUNK), jnp.float32),
            pltpu.VMEM((CHUNK * 16,), jnp.float32),
            pltpu.VMEM((ROWS_PER_W,), jnp.float32),
            pltpu.VMEM((16,), jnp.float32),
            pltpu.SemaphoreType.DMA,
            pltpu.SemaphoreType.DMA,
            pltpu.SemaphoreType.DMA,
        ],
    )(idx, wso, wlin, bias16)


def kernel(sparse_x, W_lin, W_so, bias):
    offsets = jnp.arange(F, dtype=sparse_x.dtype) * FIELD
    x = sparse_x + offsets[None, :]
    idx = (x.reshape(NW, NCHUNK, CHUNK, F)
            .transpose(0, 1, 3, 2)
            .reshape(NW, NCHUNK, F * CHUNK))
    wlin1 = W_lin.T[0]
    wt = W_so.T
    wtail = W_so[TOTAL - 128:].T
    wso_rm = _tr_call(wt, wtail).reshape(TOTAL, K)
    bias16 = jnp.broadcast_to(bias, (16,))
    return _fm_call(idx, wso_rm, wlin1, bias16)
